# baseline (speedup 1.0000x reference)
import functools

import jax
import jax.numpy as jnp
from jax import lax
from jax.experimental import pallas as pl
from jax.experimental.pallas import tpu as pltpu
from jax.experimental.pallas import tpu_sc as plsc

_VOCAB = 100000
_C = 64
_B = 32
_T = 8
_NTOK = _B * _T

_NC = 2
_NS = 16
_NW = _NC * _NS
_RPW = _NTOK // _NW


_TOK_PER_SCS = _NTOK // _NC


_NSEM = 8
_TOK_PER_SEM = _TOK_PER_SCS // _NSEM


def _sc_gather_body(tidx_hbm, tok_hbm, xs_hbm, tidx_s, sems):
    cid = lax.axis_index("c")
    base = cid * _TOK_PER_SCS
    pltpu.sync_copy(tidx_hbm.at[pl.ds(base, _TOK_PER_SCS)], tidx_s)

    for k in range(_NSEM):
        def fire(j, carry, k=k):
            i = j * _NSEM + k
            row_base = pl.multiple_of(tidx_s[i] * 8, 8)
            pltpu.async_copy(
                tok_hbm.at[pl.ds(row_base, 8)], xs_hbm.at[base + i], sems.at[k]
            )
            return carry

        lax.fori_loop(0, _TOK_PER_SEM, fire, 0)
    for k in range(_NSEM):
        slab = xs_hbm.at[pl.ds(base + k * _TOK_PER_SEM, _TOK_PER_SEM)]
        pltpu.make_async_copy(slab, slab, sems.at[k]).wait()


_sc_gather = functools.partial(
    pl.kernel,
    mesh=plsc.ScalarSubcoreMesh(axis_name="c", num_cores=_NC),
    out_type=jax.ShapeDtypeStruct((_NTOK, 8, _C), jnp.float32),
    scratch_types=[
        pltpu.SMEM((_TOK_PER_SCS,), jnp.int32),
        pltpu.SemaphoreType.DMA((_NSEM,)),
    ],
)(_sc_gather_body)


_N_TILE = 16384


def _mm_body(xs_ref, oh_ref, posb_ref, w_ref, b_ref, o_ref, x_scratch):
    @pl.when(pl.program_id(0) == 0)
    def _():
        xsel = jnp.sum(xs_ref[...] * oh_ref[...], axis=1)
        x_scratch[...] = xsel + posb_ref[...]

    o_ref[...] = (
        jnp.dot(x_scratch[...], w_ref[...], preferred_element_type=jnp.float32)
        + b_ref[...]
    )


def _head(xs, oh, posb, W, b2):
    grid = (pl.cdiv(_VOCAB, _N_TILE),)
    return pl.pallas_call(
        _mm_body,
        grid=grid,
        in_specs=[
            pl.BlockSpec((_NTOK, 8, _C), lambda i: (0, 0, 0)),
            pl.BlockSpec((_NTOK, 8, 1), lambda i: (0, 0, 0)),
            pl.BlockSpec((_NTOK, _C), lambda i: (0, 0)),
            pl.BlockSpec((_C, _N_TILE), lambda i: (0, i)),
            pl.BlockSpec((1, _N_TILE), lambda i: (0, i)),
        ],
        out_specs=pl.BlockSpec((_NTOK, _N_TILE), lambda i: (0, i)),
        out_shape=jax.ShapeDtypeStruct((_NTOK, _VOCAB), jnp.float32),
        scratch_shapes=[pltpu.VMEM((_NTOK, _C), jnp.float32)],
        compiler_params=pltpu.CompilerParams(
            dimension_semantics=("arbitrary",),
        ),
    )(xs, oh, posb, W, b2)


def kernel(idx, tok_table, pos_table, W, b):
    idx_flat = idx.reshape(-1).astype(jnp.int32)
    xs = _sc_gather(idx_flat >> 3, tok_table)
    oh = (
        (idx_flat[:, None] & 7) == jnp.arange(8, dtype=jnp.int32)[None, :]
    ).astype(jnp.float32)[:, :, None]
    posb = jnp.tile(pos_table[:_T], (_B, 1))
    logits = _head(xs, oh, posb, W, b.reshape(1, -1))
    return logits.reshape(_B, _T, _VOCAB)


# speedup vs baseline: 1.0366x; 1.0366x over previous
import functools

import jax
import jax.numpy as jnp
from jax import lax
from jax.experimental import pallas as pl
from jax.experimental.pallas import tpu as pltpu
from jax.experimental.pallas import tpu_sc as plsc

_VOCAB = 100000
_C = 64
_B = 32
_T = 8
_NTOK = _B * _T

_NC = 2
_NS = 16
_NW = _NC * _NS
_RPW = _NTOK // _NW


def _sc_gather_body(pidx_hbm, pairs_hbm, xp_hbm, pidx_v, rows_v, sem):
    wid = lax.axis_index("s") * _NC + lax.axis_index("c")
    base = wid * _RPW
    pltpu.sync_copy(pidx_hbm.at[pl.ds(base, _RPW)], pidx_v)
    pltpu.async_copy(pairs_hbm.at[pidx_v], rows_v, sem).wait()
    pltpu.sync_copy(rows_v, xp_hbm.at[pl.ds(base, _RPW)])


_sc_gather = functools.partial(
    pl.kernel,
    mesh=plsc.VectorSubcoreMesh(core_axis_name="c", subcore_axis_name="s"),
    out_type=jax.ShapeDtypeStruct((_NTOK, 2 * _C), jnp.float32),
    scratch_types=[
        pltpu.VMEM((_RPW,), jnp.int32),
        pltpu.VMEM((_RPW, 2 * _C), jnp.float32),
        pltpu.SemaphoreType.DMA,
    ],
)(_sc_gather_body)


_N_TILE = 16384


def _mm_body(xp_ref, oha_ref, posb_ref, w_ref, b_ref, o_ref, x_scratch):
    @pl.when(pl.program_id(0) == 0)
    def _():
        oha = oha_ref[...]
        x_scratch[...] = (
            xp_ref[:, : _C] * oha
            + xp_ref[:, _C :] * (1.0 - oha)
            + posb_ref[...]
        )

    o_ref[...] = (
        jnp.dot(x_scratch[...], w_ref[...], preferred_element_type=jnp.float32)
        + b_ref[...]
    )


def _head(xp, oha, posb, W, b2):
    grid = (pl.cdiv(_VOCAB, _N_TILE),)
    return pl.pallas_call(
        _mm_body,
        grid=grid,
        in_specs=[
            pl.BlockSpec((_NTOK, 2 * _C), lambda i: (0, 0)),
            pl.BlockSpec((_NTOK, 1), lambda i: (0, 0)),
            pl.BlockSpec((_NTOK, _C), lambda i: (0, 0)),
            pl.BlockSpec((_C, _N_TILE), lambda i: (0, i)),
            pl.BlockSpec((1, _N_TILE), lambda i: (0, i)),
        ],
        out_specs=pl.BlockSpec((_NTOK, _N_TILE), lambda i: (0, i)),
        out_shape=jax.ShapeDtypeStruct((_NTOK, _VOCAB), jnp.float32),
        scratch_shapes=[pltpu.VMEM((_NTOK, _C), jnp.float32)],
        compiler_params=pltpu.CompilerParams(
            dimension_semantics=("arbitrary",),
        ),
    )(xp, oha, posb, W, b2)


def kernel(idx, tok_table, pos_table, W, b):
    idx_flat = idx.reshape(-1).astype(jnp.int32)
    pairs = tok_table.reshape(_VOCAB // 2, 2 * _C)
    xp = _sc_gather(idx_flat >> 1, pairs)
    oha = ((idx_flat & 1) == 0).astype(jnp.float32)[:, None]
    posb = jnp.tile(pos_table[:_T], (_B, 1))
    logits = _head(xp, oha, posb, W, b.reshape(1, -1))
    return logits.reshape(_B, _T, _VOCAB)
